# Initial kernel scaffold; baseline (speedup 1.0000x reference)
import jax
import jax.numpy as jnp
from jax.experimental import pallas as pl


def kernel(hidden_states, gate_w, W1, b1, W2, b2):
    raise NotImplementedError("write your pallas kernel here")


# speedup vs baseline: 3.6063x; 3.6063x over previous
import functools
import math

import jax
import jax.numpy as jnp
from jax import lax
from jax.experimental import pallas as pl
from jax.experimental.pallas import tpu as pltpu


def _router_body(chunk, x_ref, gw_ref, route_ref, oh_ref):
    x = x_ref[:]
    gw = gw_ref[:]
    N = x.shape[0]
    E = gw.shape[0]
    logits = lax.dot_general(x, gw, (((1,), (1,)), ((), ())),
                             preferred_element_type=jnp.float32)
    m = jnp.max(logits, axis=1, keepdims=True)
    p = jnp.exp(logits - m)
    gates = p / jnp.sum(p, axis=1, keepdims=True)
    gate = jnp.max(gates, axis=1, keepdims=True)
    iota_e = lax.broadcasted_iota(jnp.int32, (N, E), 1).astype(jnp.float32)
    e_idx = jnp.min(jnp.where(gates >= gate, iota_e, jnp.float32(E)),
                    axis=1, keepdims=True)
    oh_ref[:] = (iota_e == e_idx).astype(jnp.float32)
    route_ref[:, 0:1] = e_idx
    route_ref[:, 2:3] = gate

    def body(i, _):
        base = i * chunk
        r_i = lax.broadcasted_iota(jnp.int32, (chunk, N), 0) + base
        c_i = lax.broadcasted_iota(jnp.int32, (chunk, N), 1)
        tri = (c_i <= r_i).astype(jnp.float32)
        csum = lax.dot_general(tri, oh_ref[:], (((1,), (0,)), ((), ())),
                               preferred_element_type=jnp.float32)
        oh_c = oh_ref[pl.ds(base, chunk), :]
        pos = jnp.sum(csum * oh_c, axis=1, keepdims=True) - 1.0
        route_ref[pl.ds(base, chunk), 1:2] = pos
        return 0

    lax.fori_loop(0, N // chunk, body, 0)


def _expert_body(C, x_ref, route_ref, w1_ref, b1_ref, w2_ref, b2_ref, out_ref):
    e = pl.program_id(0)
    ef = lax.convert_element_type(e, jnp.float32)
    N = x_ref.shape[0]
    ecol = route_ref[:, 0:1]
    pcol = route_ref[:, 1:2]
    gcol = route_ref[:, 2:3]
    keep = (ecol == ef) & (pcol < jnp.float32(C))
    iota_c = lax.broadcasted_iota(jnp.int32, (N, C), 1).astype(jnp.float32)
    P = jnp.where(keep & (pcol == iota_c), 1.0, 0.0)
    xe = lax.dot_general(P, x_ref[:], (((0,), (0,)), ((), ())),
                         preferred_element_type=jnp.float32)
    h = lax.dot_general(xe, w1_ref[0], (((1,), (1,)), ((), ())),
                        preferred_element_type=jnp.float32) + b1_ref[0]
    h = 0.5 * h * (1.0 + lax.erf(h * 0.7071067811865476))
    o = lax.dot_general(h, w2_ref[0], (((1,), (1,)), ((), ())),
                        preferred_element_type=jnp.float32) + b2_ref[0]

    @pl.when(e == 0)
    def _():
        out_ref[:] = jnp.zeros_like(out_ref)

    out_ref[:] += lax.dot_general(P * gcol, o, (((1,), (0,)), ((), ())),
                                  preferred_element_type=jnp.float32)


def kernel(hidden_states, gate_w, W1, b1, W2, b2):
    Bs, Ts, D = hidden_states.shape
    N = Bs * Ts
    E, H = W1.shape[0], W1.shape[1]
    C = math.ceil(2.0 * N / E)
    flat = hidden_states.reshape(N, D)

    route = pl.pallas_call(
        functools.partial(_router_body, 128),
        out_shape=jax.ShapeDtypeStruct((N, 128), jnp.float32),
        scratch_shapes=[pltpu.VMEM((N, E), jnp.float32)],
    )(flat, gate_w)

    final = pl.pallas_call(
        functools.partial(_expert_body, C),
        grid=(E,),
        in_specs=[
            pl.BlockSpec((N, D), lambda e: (0, 0)),
            pl.BlockSpec((N, 128), lambda e: (0, 0)),
            pl.BlockSpec((1, H, D), lambda e: (e, 0, 0)),
            pl.BlockSpec((1, 1, H), lambda e: (e, 0, 0)),
            pl.BlockSpec((1, D, H), lambda e: (e, 0, 0)),
            pl.BlockSpec((1, 1, D), lambda e: (e, 0, 0)),
        ],
        out_specs=pl.BlockSpec((N, D), lambda e: (0, 0)),
        out_shape=jax.ShapeDtypeStruct((N, D), jnp.float32),
    )(flat, route, W1, b1.reshape(E, 1, H), W2, b2.reshape(E, 1, D))

    aux_loss = jnp.asarray(0.0, dtype=jnp.float32)
    return final.reshape(Bs, Ts, D), aux_loss


# speedup vs baseline: 3.7843x; 1.0494x over previous
import functools
import math

import jax
import jax.numpy as jnp
from jax import lax
from jax.experimental import pallas as pl
from jax.experimental.pallas import tpu as pltpu


def _route(chunk, x_ref, gw_ref, route_ref, oh_ref):
    x = x_ref[:]
    gw = gw_ref[:]
    N = x.shape[0]
    E = gw.shape[0]
    logits = lax.dot_general(x, gw, (((1,), (1,)), ((), ())),
                             preferred_element_type=jnp.float32)
    m = jnp.max(logits, axis=1, keepdims=True)
    p = jnp.exp(logits - m)
    gates = p / jnp.sum(p, axis=1, keepdims=True)
    gate = jnp.max(gates, axis=1, keepdims=True)
    iota_e = lax.broadcasted_iota(jnp.int32, (N, E), 1).astype(jnp.float32)
    e_idx = jnp.min(jnp.where(gates >= gate, iota_e, jnp.float32(E)),
                    axis=1, keepdims=True)
    oh_ref[:] = (iota_e == e_idx).astype(jnp.float32)
    route_ref[:, 0:1] = e_idx
    route_ref[:, 2:3] = gate

    def body(i, _):
        base = i * chunk
        r_i = lax.broadcasted_iota(jnp.int32, (chunk, N), 0) + base
        c_i = lax.broadcasted_iota(jnp.int32, (chunk, N), 1)
        tri = (c_i <= r_i).astype(jnp.float32)
        csum = lax.dot_general(tri, oh_ref[:], (((1,), (0,)), ((), ())),
                               preferred_element_type=jnp.float32)
        oh_c = oh_ref[pl.ds(base, chunk), :]
        pos = jnp.sum(csum * oh_c, axis=1, keepdims=True) - 1.0
        route_ref[pl.ds(base, chunk), 1:2] = pos
        return 0

    lax.fori_loop(0, N // chunk, body, 0)


def _body(C, chunk, x_ref, gw_ref, w1_ref, b1_ref, w2_ref, b2_ref, out_ref,
          route_ref, oh_ref):
    e = pl.program_id(0)
    N = x_ref.shape[0]

    @pl.when(e == 0)
    def _():
        _route(chunk, x_ref, gw_ref, route_ref, oh_ref)
        out_ref[:] = jnp.zeros_like(out_ref)

    ef = lax.convert_element_type(e, jnp.float32)
    ecol = route_ref[:, 0:1]
    pcol = route_ref[:, 1:2]
    gcol = route_ref[:, 2:3]
    keep = (ecol == ef) & (pcol < jnp.float32(C))
    iota_c = lax.broadcasted_iota(jnp.int32, (N, C), 1).astype(jnp.float32)
    P = jnp.where(keep & (pcol == iota_c), 1.0, 0.0)
    xe = lax.dot_general(P, x_ref[:], (((0,), (0,)), ((), ())),
                         preferred_element_type=jnp.float32)
    h = lax.dot_general(xe, w1_ref[0], (((1,), (1,)), ((), ())),
                        preferred_element_type=jnp.float32) + b1_ref[0]
    h = 0.5 * h * (1.0 + lax.erf(h * 0.7071067811865476))
    o = lax.dot_general(h, w2_ref[0], (((1,), (1,)), ((), ())),
                        preferred_element_type=jnp.float32) + b2_ref[0]
    out_ref[:] += lax.dot_general(P * gcol, o, (((1,), (0,)), ((), ())),
                                  preferred_element_type=jnp.float32)


def kernel(hidden_states, gate_w, W1, b1, W2, b2):
    Bs, Ts, D = hidden_states.shape
    N = Bs * Ts
    E, H = W1.shape[0], W1.shape[1]
    C = math.ceil(2.0 * N / E)
    flat = hidden_states.reshape(N, D)

    final = pl.pallas_call(
        functools.partial(_body, C, 128),
        grid=(E,),
        in_specs=[
            pl.BlockSpec((N, D), lambda e: (0, 0)),
            pl.BlockSpec((E, D), lambda e: (0, 0)),
            pl.BlockSpec((1, H, D), lambda e: (e, 0, 0)),
            pl.BlockSpec((1, 1, H), lambda e: (e, 0, 0)),
            pl.BlockSpec((1, D, H), lambda e: (e, 0, 0)),
            pl.BlockSpec((1, 1, D), lambda e: (e, 0, 0)),
        ],
        out_specs=pl.BlockSpec((N, D), lambda e: (0, 0)),
        out_shape=jax.ShapeDtypeStruct((N, D), jnp.float32),
        scratch_shapes=[
            pltpu.VMEM((N, 128), jnp.float32),
            pltpu.VMEM((N, E), jnp.float32),
        ],
    )(flat, gate_w, W1, b1.reshape(E, 1, H), W2, b2.reshape(E, 1, D))

    aux_loss = jnp.asarray(0.0, dtype=jnp.float32)
    return final.reshape(Bs, Ts, D), aux_loss
